# Initial kernel scaffold; baseline (speedup 1.0000x reference)
import jax
import jax.numpy as jnp
from jax.experimental import pallas as pl


def kernel(input_ids, word_emb, pos_emb, ln_gamma, ln_beta):
    raise NotImplementedError("write your pallas kernel here")


# speedup vs baseline: 2.9097x; 2.9097x over previous
import functools

import jax
import jax.numpy as jnp
from jax import lax
from jax.experimental import pallas as pl
from jax.experimental.pallas import tpu as pltpu


def _round_up(x, m):
    return (x + m - 1) // m * m


def _emb_ln_kernel(ids_ref,
                   word_hbm,
                   pos_ref,
                   gamma_ref,
                   beta_ref,
                   out_ref,
                   wbuf,
                   sem,
                   *, tq, seq_p, eps):
    b = pl.program_id(0)
    j = pl.program_id(1)
    n_j = pl.num_programs(1)
    slot = lax.rem(j, 2)

    def start_gather(tile_j, slot_):
        base = b * seq_p + tile_j * tq
        for r in range(tq):
            tok = ids_ref[base + r]
            pltpu.make_async_copy(
                word_hbm.at[pl.ds(tok, 1), :],
                wbuf.at[slot_, pl.ds(r, 1), :],
                sem.at[slot_],
            ).start()

    @pl.when(j == 0)
    def _():
        start_gather(0, slot)

    @pl.when(j + 1 < n_j)
    def _():
        start_gather(j + 1, 1 - slot)

    pltpu.make_async_copy(
        word_hbm.at[pl.ds(0, tq), :],
        wbuf.at[slot],
        sem.at[slot],
    ).wait()

    row0 = pl.multiple_of(j * tq, 8)
    x = wbuf[slot] + pos_ref[pl.ds(row0, tq), :]
    mean = jnp.mean(x, axis=-1, keepdims=True)
    xc = x - mean
    var = jnp.mean(xc * xc, axis=-1, keepdims=True)
    scale = lax.rsqrt(var + eps) * gamma_ref[...]
    out_ref[...] = xc * scale + beta_ref[...]


def _embeddings(input_ids, word_emb, pos_emb, ln_gamma, ln_beta,
                eps=1e-12, tq=256):
    bs, seq = input_ids.shape
    vocab, d = word_emb.shape

    tq = min(tq, _round_up(seq, 8))
    seq_p = _round_up(seq, tq)
    n_j = seq_p // tq

    ids = input_ids.astype(jnp.int32)
    if seq_p != seq:
        ids = jnp.pad(ids, ((0, 0), (0, seq_p - seq)))
    ids_flat = ids.reshape(bs * seq_p)

    pos_tab = pos_emb
    if pos_tab.shape[0] < seq_p:
        pos_tab = jnp.pad(pos_tab, ((0, seq_p - pos_tab.shape[0]), (0, 0)))
    elif pos_tab.shape[0] > seq_p:
        pos_tab = pos_tab[:seq_p]

    gamma2 = ln_gamma.reshape(1, d)
    beta2 = ln_beta.reshape(1, d)

    grid_spec = pltpu.PrefetchScalarGridSpec(
        num_scalar_prefetch=1,
        grid=(bs, n_j),
        in_specs=[
            pl.BlockSpec(memory_space=pl.ANY),
            pl.BlockSpec((seq_p, d), lambda b, j, ids_smem: (0, 0)),
            pl.BlockSpec((1, d), lambda b, j, ids_smem: (0, 0)),
            pl.BlockSpec((1, d), lambda b, j, ids_smem: (0, 0)),
        ],
        out_specs=pl.BlockSpec((tq, d), lambda b, j, ids_smem: (b * n_j + j, 0)),
        scratch_shapes=[
            pltpu.VMEM((2, tq, d), jnp.float32),
            pltpu.SemaphoreType.DMA((2,)),
        ],
    )

    out_flat = pl.pallas_call(
        functools.partial(_emb_ln_kernel, tq=tq, seq_p=seq_p, eps=eps),
        grid_spec=grid_spec,
        out_shape=jax.ShapeDtypeStruct((bs * seq_p, d), jnp.float32),
        compiler_params=pltpu.CompilerParams(
            dimension_semantics=("parallel", "arbitrary"),
            disable_bounds_checks=True,
        ),
    )(ids_flat, word_emb, pos_tab, gamma2, beta2)

    out = out_flat.reshape(bs, seq_p, d)
    if seq_p != seq:
        out = out[:, :seq, :]
    return out


def kernel(input_ids, word_emb, pos_emb, ln_gamma, ln_beta):
    return _embeddings(input_ids, word_emb, pos_emb, ln_gamma, ln_beta)


# speedup vs baseline: 2.9444x; 1.0119x over previous
import functools

import jax
import jax.numpy as jnp
from jax import lax
from jax.experimental import pallas as pl
from jax.experimental.pallas import tpu as pltpu


def _round_up(x, m):
    return (x + m - 1) // m * m


def _emb_ln_kernel(ids_ref,
                   word_hbm,
                   pos_ref,
                   gamma_ref,
                   beta_ref,
                   out_ref,
                   wbuf,
                   sem,
                   *, tq, seq_p, eps):
    b = pl.program_id(0)
    j = pl.program_id(1)
    n_j = pl.num_programs(1)
    slot = lax.rem(j, 2)

    def start_gather(tile_j, slot_):
        base = b * seq_p + tile_j * tq
        for r in range(tq):
            tok = ids_ref[base + r]
            pltpu.make_async_copy(
                word_hbm.at[pl.ds(tok, 1), :],
                wbuf.at[slot_, pl.ds(r, 1), :],
                sem.at[slot_],
            ).start(priority=r % 2)

    @pl.when(j == 0)
    def _():
        start_gather(0, slot)

    @pl.when(j + 1 < n_j)
    def _():
        start_gather(j + 1, 1 - slot)

    pltpu.make_async_copy(
        word_hbm.at[pl.ds(0, tq), :],
        wbuf.at[slot],
        sem.at[slot],
    ).wait()

    row0 = pl.multiple_of(j * tq, 8)
    x = wbuf[slot] + pos_ref[pl.ds(row0, tq), :]
    mean = jnp.mean(x, axis=-1, keepdims=True)
    xc = x - mean
    var = jnp.mean(xc * xc, axis=-1, keepdims=True)
    scale = lax.rsqrt(var + eps) * gamma_ref[...]
    out_ref[...] = xc * scale + beta_ref[...]


def _embeddings(input_ids, word_emb, pos_emb, ln_gamma, ln_beta,
                eps=1e-12, tq=256):
    bs, seq = input_ids.shape
    vocab, d = word_emb.shape

    tq = min(tq, _round_up(seq, 8))
    seq_p = _round_up(seq, tq)
    n_j = seq_p // tq

    ids = input_ids.astype(jnp.int32)
    if seq_p != seq:
        ids = jnp.pad(ids, ((0, 0), (0, seq_p - seq)))
    ids_flat = ids.reshape(bs * seq_p)

    pos_tab = pos_emb
    if pos_tab.shape[0] < seq_p:
        pos_tab = jnp.pad(pos_tab, ((0, seq_p - pos_tab.shape[0]), (0, 0)))
    elif pos_tab.shape[0] > seq_p:
        pos_tab = pos_tab[:seq_p]

    gamma2 = ln_gamma.reshape(1, d)
    beta2 = ln_beta.reshape(1, d)

    grid_spec = pltpu.PrefetchScalarGridSpec(
        num_scalar_prefetch=1,
        grid=(bs, n_j),
        in_specs=[
            pl.BlockSpec(memory_space=pl.ANY),
            pl.BlockSpec((seq_p, d), lambda b, j, ids_smem: (0, 0)),
            pl.BlockSpec((1, d), lambda b, j, ids_smem: (0, 0)),
            pl.BlockSpec((1, d), lambda b, j, ids_smem: (0, 0)),
        ],
        out_specs=pl.BlockSpec((tq, d), lambda b, j, ids_smem: (b * n_j + j, 0)),
        scratch_shapes=[
            pltpu.VMEM((2, tq, d), jnp.float32),
            pltpu.SemaphoreType.DMA((2,)),
        ],
    )

    out_flat = pl.pallas_call(
        functools.partial(_emb_ln_kernel, tq=tq, seq_p=seq_p, eps=eps),
        grid_spec=grid_spec,
        out_shape=jax.ShapeDtypeStruct((bs * seq_p, d), jnp.float32),
        compiler_params=pltpu.CompilerParams(
            dimension_semantics=("parallel", "arbitrary"),
            disable_bounds_checks=True,
        ),
    )(ids_flat, word_emb, pos_tab, gamma2, beta2)

    out = out_flat.reshape(bs, seq_p, d)
    if seq_p != seq:
        out = out[:, :seq, :]
    return out


def kernel(input_ids, word_emb, pos_emb, ln_gamma, ln_beta):
    return _embeddings(input_ids, word_emb, pos_emb, ln_gamma, ln_beta)


# speedup vs baseline: 3.4315x; 1.1654x over previous
import functools

import jax
import jax.numpy as jnp
from jax import lax
from jax.experimental import pallas as pl
from jax.experimental.pallas import tpu as pltpu


def _round_up(x, m):
    return (x + m - 1) // m * m


def _emb_ln_kernel(ids_ref,
                   word_hbm,
                   pos_ref,
                   gamma_ref,
                   beta_ref,
                   out_ref,
                   wbuf,
                   sem,
                   *, tq, seq_p, eps):
    c = pl.program_id(0)
    b = pl.program_id(1)
    j = pl.program_id(2)
    n_b = pl.num_programs(1)
    n_j = pl.num_programs(2)
    t = b * n_j + j
    slot = lax.rem(t, 2)

    def start_gather(row, tile_j, slot_):
        base = row * seq_p + tile_j * tq
        for r in range(tq):
            tok = ids_ref[base + r]
            pltpu.make_async_copy(
                word_hbm.at[pl.ds(tok, 1), :],
                wbuf.at[slot_, pl.ds(r, 1), :],
                sem.at[slot_],
            ).start(priority=r % 2)

    @pl.when((b == 0) & (j == 0))
    def _():
        start_gather(c * n_b, 0, slot)

    @pl.when(t + 1 < n_b * n_j)
    def _():
        nxt_j = j + 1
        wrap = nxt_j == n_j
        nxt_b = b + wrap.astype(jnp.int32)
        nxt_j = jnp.where(wrap, 0, nxt_j)
        start_gather(c * n_b + nxt_b, nxt_j, 1 - slot)

    pltpu.make_async_copy(
        word_hbm.at[pl.ds(0, tq), :],
        wbuf.at[slot],
        sem.at[slot],
    ).wait()

    row0 = pl.multiple_of(j * tq, 8)
    x = wbuf[slot] + pos_ref[pl.ds(row0, tq), :]
    mean = jnp.mean(x, axis=-1, keepdims=True)
    xc = x - mean
    var = jnp.mean(xc * xc, axis=-1, keepdims=True)
    scale = lax.rsqrt(var + eps) * gamma_ref[...]
    out_ref[...] = xc * scale + beta_ref[...]


def _embeddings(input_ids, word_emb, pos_emb, ln_gamma, ln_beta,
                eps=1e-12, tq=256):
    bs, seq = input_ids.shape
    vocab, d = word_emb.shape

    tq = min(tq, _round_up(seq, 8))
    seq_p = _round_up(seq, tq)
    n_j = seq_p // tq
    n_c = 2 if bs % 2 == 0 else 1
    n_b = bs // n_c

    ids = input_ids.astype(jnp.int32)
    if seq_p != seq:
        ids = jnp.pad(ids, ((0, 0), (0, seq_p - seq)))
    ids_flat = ids.reshape(bs * seq_p)

    pos_tab = pos_emb
    if pos_tab.shape[0] < seq_p:
        pos_tab = jnp.pad(pos_tab, ((0, seq_p - pos_tab.shape[0]), (0, 0)))
    elif pos_tab.shape[0] > seq_p:
        pos_tab = pos_tab[:seq_p]

    gamma2 = ln_gamma.reshape(1, d)
    beta2 = ln_beta.reshape(1, d)

    grid_spec = pltpu.PrefetchScalarGridSpec(
        num_scalar_prefetch=1,
        grid=(n_c, n_b, n_j),
        in_specs=[
            pl.BlockSpec(memory_space=pl.ANY),
            pl.BlockSpec((seq_p, d), lambda c, b, j, ids_smem: (0, 0)),
            pl.BlockSpec((1, d), lambda c, b, j, ids_smem: (0, 0)),
            pl.BlockSpec((1, d), lambda c, b, j, ids_smem: (0, 0)),
        ],
        out_specs=pl.BlockSpec(
            (tq, d),
            lambda c, b, j, ids_smem: ((c * n_b + b) * n_j + j, 0)),
        scratch_shapes=[
            pltpu.VMEM((2, tq, d), jnp.float32),
            pltpu.SemaphoreType.DMA((2,)),
        ],
    )

    out_flat = pl.pallas_call(
        functools.partial(_emb_ln_kernel, tq=tq, seq_p=seq_p, eps=eps),
        grid_spec=grid_spec,
        out_shape=jax.ShapeDtypeStruct((bs * seq_p, d), jnp.float32),
        compiler_params=pltpu.CompilerParams(
            dimension_semantics=("parallel", "arbitrary", "arbitrary"),
            disable_bounds_checks=True,
        ),
    )(ids_flat, word_emb, pos_tab, gamma2, beta2)

    out = out_flat.reshape(bs, seq_p, d)
    if seq_p != seq:
        out = out[:, :seq, :]
    return out


def kernel(input_ids, word_emb, pos_emb, ln_gamma, ln_beta):
    return _embeddings(input_ids, word_emb, pos_emb, ln_gamma, ln_beta)


# speedup vs baseline: 3.5817x; 1.0438x over previous
import functools

import jax
import jax.numpy as jnp
from jax import lax
from jax.experimental import pallas as pl
from jax.experimental.pallas import tpu as pltpu


def _round_up(x, m):
    return (x + m - 1) // m * m


def _emb_ln_kernel(ids_ref,
                   word_hbm,
                   pos_ref,
                   gamma_ref,
                   beta_ref,
                   out_ref,
                   wbuf,
                   sem,
                   *, tq, seq_p, eps):
    c = pl.program_id(0)
    b = pl.program_id(1)
    j = pl.program_id(2)
    n_b = pl.num_programs(1)
    n_j = pl.num_programs(2)
    t = b * n_j + j
    slot = lax.rem(t, 2)

    def start_gather(row, tile_j, slot_):
        base = row * seq_p + tile_j * tq
        for r in range(tq):
            tok = ids_ref[base + r]
            pltpu.make_async_copy(
                word_hbm.at[pl.ds(tok, 1), :],
                wbuf.at[slot_, pl.ds(r, 1), :],
                sem.at[slot_],
            ).start(priority=r % 2)

    @pl.when((b == 0) & (j == 0))
    def _():
        start_gather(c * n_b, 0, slot)

    @pl.when(t + 1 < n_b * n_j)
    def _():
        nxt_j = j + 1
        wrap = nxt_j == n_j
        nxt_b = b + wrap.astype(jnp.int32)
        nxt_j = jnp.where(wrap, 0, nxt_j)
        start_gather(c * n_b + nxt_b, nxt_j, 1 - slot)

    pltpu.make_async_copy(
        word_hbm.at[pl.ds(0, tq), :],
        wbuf.at[slot],
        sem.at[slot],
    ).wait()

    row0 = pl.multiple_of(j * tq, 8)
    x = wbuf[slot] + pos_ref[pl.ds(row0, tq), :]
    mean = jnp.mean(x, axis=-1, keepdims=True)
    xc = x - mean
    var = jnp.mean(xc * xc, axis=-1, keepdims=True)
    scale = lax.rsqrt(var + eps) * gamma_ref[...]
    out_ref[...] = xc * scale + beta_ref[...]


def _embeddings(input_ids, word_emb, pos_emb, ln_gamma, ln_beta,
                eps=1e-12, tq=512):
    bs, seq = input_ids.shape
    vocab, d = word_emb.shape

    tq = min(tq, _round_up(seq, 8))
    seq_p = _round_up(seq, tq)
    n_j = seq_p // tq
    n_c = 2 if bs % 2 == 0 else 1
    n_b = bs // n_c

    ids = input_ids.astype(jnp.int32)
    if seq_p != seq:
        ids = jnp.pad(ids, ((0, 0), (0, seq_p - seq)))
    ids_flat = ids.reshape(bs * seq_p)

    pos_tab = pos_emb
    if pos_tab.shape[0] < seq_p:
        pos_tab = jnp.pad(pos_tab, ((0, seq_p - pos_tab.shape[0]), (0, 0)))
    elif pos_tab.shape[0] > seq_p:
        pos_tab = pos_tab[:seq_p]

    gamma2 = ln_gamma.reshape(1, d)
    beta2 = ln_beta.reshape(1, d)

    grid_spec = pltpu.PrefetchScalarGridSpec(
        num_scalar_prefetch=1,
        grid=(n_c, n_b, n_j),
        in_specs=[
            pl.BlockSpec(memory_space=pl.ANY),
            pl.BlockSpec((seq_p, d), lambda c, b, j, ids_smem: (0, 0)),
            pl.BlockSpec((1, d), lambda c, b, j, ids_smem: (0, 0)),
            pl.BlockSpec((1, d), lambda c, b, j, ids_smem: (0, 0)),
        ],
        out_specs=pl.BlockSpec(
            (tq, d),
            lambda c, b, j, ids_smem: ((c * n_b + b) * n_j + j, 0)),
        scratch_shapes=[
            pltpu.VMEM((2, tq, d), jnp.float32),
            pltpu.SemaphoreType.DMA((2,)),
        ],
    )

    out_flat = pl.pallas_call(
        functools.partial(_emb_ln_kernel, tq=tq, seq_p=seq_p, eps=eps),
        grid_spec=grid_spec,
        out_shape=jax.ShapeDtypeStruct((bs * seq_p, d), jnp.float32),
        compiler_params=pltpu.CompilerParams(
            dimension_semantics=("parallel", "arbitrary", "arbitrary"),
            disable_bounds_checks=True,
        ),
    )(ids_flat, word_emb, pos_tab, gamma2, beta2)

    out = out_flat.reshape(bs, seq_p, d)
    if seq_p != seq:
        out = out[:, :seq, :]
    return out


def kernel(input_ids, word_emb, pos_emb, ln_gamma, ln_beta):
    return _embeddings(input_ids, word_emb, pos_emb, ln_gamma, ln_beta)
